# baseline (speedup 1.0000x reference)
import jax
import jax.numpy as jnp
from jax.experimental import pallas as pl
from jax.experimental.pallas import tpu as pltpu

_N = 8192
_K = 4096
_ROWS = 128
_BLOCK_ROWS = 8


def _sdd_block(x_ref, o_ref):
    int_max = jnp.int32(2**31 - 1)
    int_min = jnp.int32(-(2**31))
    x = x_ref[...]
    xz = x + 0.0
    b = jax.lax.bitcast_convert_type(xz, jnp.int32)
    key = jnp.where(b >= 0, b, int_max - b)

    cnt_pos = jnp.sum((key >= 0).astype(jnp.int32), axis=1)
    prefix = jnp.where(cnt_pos >= _K, jnp.int32(0), int_min)
    cnt0 = jnp.where(cnt_pos >= _K, cnt_pos, jnp.int32(_N))

    def cond(state):
        i, _, cntp = state
        return jnp.logical_and(i < 31, jnp.any(cntp > _K))

    def body(state):
        i, prefix, cntp = state
        bit = jnp.left_shift(jnp.int32(1), jnp.int32(30) - i)
        cand = prefix + bit
        cnt = jnp.sum((key >= cand[:, None]).astype(jnp.int32), axis=1)
        take = cnt >= _K
        return (i + jnp.int32(1),
                jnp.where(take, cand, prefix),
                jnp.where(take, cnt, cntp))

    _, prefix, _ = jax.lax.while_loop(
        cond, body, (jnp.int32(0), prefix, cnt0))

    mask = key >= prefix[:, None]
    o_ref[...] = jnp.where(mask, x, jnp.float32(0.0))


def kernel(x):
    return pl.pallas_call(
        _sdd_block,
        out_shape=jax.ShapeDtypeStruct(x.shape, x.dtype),
        grid=(_ROWS // _BLOCK_ROWS,),
        in_specs=[pl.BlockSpec((_BLOCK_ROWS, _N), lambda i: (i, 0))],
        out_specs=pl.BlockSpec((_BLOCK_ROWS, _N), lambda i: (i, 0)),
        compiler_params=pltpu.CompilerParams(
            dimension_semantics=("parallel",)
        ),
    )(x)


# speedup vs baseline: 1.2962x; 1.2962x over previous
import jax
import jax.numpy as jnp
from jax.experimental import pallas as pl
from jax.experimental.pallas import tpu as pltpu

_N = 8192
_K = 4096
_ROWS = 128
_BLOCK_ROWS = 8


def _sdd_block(x_ref, o_ref):
    int_max = jnp.int32(2**31 - 1)
    int_min = jnp.int32(-(2**31))
    x = x_ref[...]
    xz = x + 0.0
    b = jax.lax.bitcast_convert_type(xz, jnp.int32)
    key = jnp.where(b >= 0, b, int_max - b)

    cnt_pos = jnp.sum((key >= 0).astype(jnp.int32), axis=1)
    prefix = jnp.where(cnt_pos >= _K, jnp.int32(0), int_min)
    cnt0 = jnp.where(cnt_pos >= _K, cnt_pos, jnp.int32(_N))

    def sweep(i, prefix, cntp):
        bit = jnp.left_shift(jnp.int32(1), jnp.int32(30) - i)
        cand = prefix + bit
        cnt = jnp.sum((key >= cand[:, None]).astype(jnp.int32), axis=1)
        take = cnt >= _K
        return jnp.where(take, cand, prefix), jnp.where(take, cnt, cntp)

    def cond(state):
        i, _, cntp = state
        return jnp.logical_and(i < 28, jnp.any(cntp > _K))

    def body(state):
        i, prefix, cntp = state
        for j in range(4):
            prefix, cntp = sweep(i + jnp.int32(j), prefix, cntp)
        return (i + jnp.int32(4), prefix, cntp)

    i, prefix, cntp = jax.lax.while_loop(
        cond, body, (jnp.int32(0), prefix, cnt0))
    for j in range(3):
        prefix, cntp = sweep(jnp.int32(28 + j), prefix, cntp)

    mask = key >= prefix[:, None]
    o_ref[...] = jnp.where(mask, x, jnp.float32(0.0))


def kernel(x):
    return pl.pallas_call(
        _sdd_block,
        out_shape=jax.ShapeDtypeStruct(x.shape, x.dtype),
        grid=(_ROWS // _BLOCK_ROWS,),
        in_specs=[pl.BlockSpec((_BLOCK_ROWS, _N), lambda i: (i, 0))],
        out_specs=pl.BlockSpec((_BLOCK_ROWS, _N), lambda i: (i, 0)),
        compiler_params=pltpu.CompilerParams(
            dimension_semantics=("parallel",)
        ),
    )(x)


# speedup vs baseline: 3.2021x; 2.4705x over previous
import jax
import jax.numpy as jnp
from jax.experimental import pallas as pl
from jax.experimental.pallas import tpu as pltpu

_N = 8192
_K = 4096
_ROWS = 128
_BLOCK_ROWS = 32


def _rowsum(c):
    while c.shape[1] > 128:
        h = c.shape[1] // 2
        c = c[:, :h] + c[:, h:]
    return jnp.sum(c, axis=1, keepdims=True)


def _sdd_block(x_ref, o_ref):
    int_max = jnp.int32(2**31 - 1)
    int_min = jnp.int32(-(2**31))
    x = x_ref[...]
    xz = x + 0.0
    b = jax.lax.bitcast_convert_type(xz, jnp.int32)
    key = jnp.where(b >= 0, b, int_max - b)

    cnt_pos = jnp.sum((key >= 0).astype(jnp.int32), axis=1, keepdims=True)
    pos = cnt_pos >= _K
    prefix = jnp.where(pos, jnp.int32(0), int_min)
    cnt0 = jnp.where(pos, cnt_pos, jnp.int32(_N))

    def sweep(i, prefix, cntp):
        bit = jnp.left_shift(jnp.int32(1), jnp.int32(30) - i)
        cand = prefix + bit
        cnt = _rowsum((key >= cand).astype(jnp.int32))
        take = cnt >= _K
        return jnp.where(take, cand, prefix), jnp.where(take, cnt, cntp)

    def cond(state):
        i, _, cntp = state
        return jnp.logical_and(i < 28, jnp.any(cntp > _K))

    def body(state):
        i, prefix, cntp = state
        for j in range(4):
            prefix, cntp = sweep(i + jnp.int32(j), prefix, cntp)
        return (i + jnp.int32(4), prefix, cntp)

    i, prefix, cntp = jax.lax.while_loop(
        cond, body, (jnp.int32(0), prefix, cnt0))
    for j in range(3):
        prefix, cntp = sweep(jnp.int32(28 + j), prefix, cntp)

    mask = key >= prefix
    o_ref[...] = jnp.where(mask, x, jnp.float32(0.0))


def kernel(x):
    return pl.pallas_call(
        _sdd_block,
        out_shape=jax.ShapeDtypeStruct(x.shape, x.dtype),
        grid=(_ROWS // _BLOCK_ROWS,),
        in_specs=[pl.BlockSpec((_BLOCK_ROWS, _N), lambda i: (i, 0))],
        out_specs=pl.BlockSpec((_BLOCK_ROWS, _N), lambda i: (i, 0)),
        compiler_params=pltpu.CompilerParams(
            dimension_semantics=("parallel",)
        ),
    )(x)
